# baseline (speedup 1.0000x reference)
import functools

import jax
import jax.numpy as jnp
from jax import lax
from jax.experimental import pallas as pl
from jax.experimental.pallas import tpu as pltpu
from jax.experimental.pallas import tpu_sc as plsc

NUM_CODES = 8192
DIM = 256
NTOK = 8192
CCOST = 0.25

BN = 1024
BK = 1024


NB = NTOK // BN
KB = NUM_CODES // BK


def _argmin_body(rowsq_ref, esq_ref, x_ref, e2_ref, idx_ref, enc_ref,
                 loss_ref, min_s, arg_s, prev_s, acc_s):
    i = pl.program_id(0)
    j = pl.program_id(1)

    @pl.when((j == 0) & (i > 0))
    def _():
        prev_s[...] = arg_s[...]

    @pl.when(j == 0)
    def _():
        min_s[...] = jnp.full((BN,), jnp.inf, jnp.float32)
        arg_s[...] = jnp.zeros((BN,), jnp.int32)

    @pl.when(i < NB)
    def _():
        e2 = e2_ref[pl.ds(j * BK, BK), :] * 2.0
        mm2 = jnp.dot(e2, x_ref[0],
                      preferred_element_type=jnp.float32)
        d = (rowsq_ref[...] + esq_ref[...]) - mm2
        lmin = jnp.min(d, axis=0)
        rows = lax.broadcasted_iota(jnp.int32, (BK, BN), 0)
        larg = jnp.min(jnp.where(d == lmin[None, :], rows, BK),
                       axis=0) + j * BK
        better = lmin < min_s[...]
        arg_s[...] = jnp.where(better, larg, arg_s[...])
        min_s[...] = jnp.where(better, lmin, min_s[...])

        @pl.when(j == KB - 1)
        def _():
            s = jnp.sum(min_s[...])
            acc_s[0, 0] = jnp.where(i == 0, s, acc_s[0, 0] + s)

    @pl.when((i == NB) & (j == 0))
    def _():
        m = acc_s[0, 0] / (NTOK * DIM)
        loss_ref[0, 0] = m + CCOST * m

    @pl.when(i > 0)
    def _():
        idv = prev_s[...]
        cols2 = lax.broadcasted_iota(jnp.int32, (BN, BK), 1) + j * BK
        enc_ref[...] = (idv[:, None] == cols2).astype(jnp.float32)

        @pl.when(j == 0)
        def _():
            idx_ref[...] = idv.reshape(1, 1, BN)


_argmin_call = pl.pallas_call(
    _argmin_body,
    grid=(NB + 1, KB),
    in_specs=[
        pl.BlockSpec((1, BN), lambda i, j: (0, jnp.minimum(i, NB - 1))),
        pl.BlockSpec((BK, 1), lambda i, j: (j, 0)),
        pl.BlockSpec((1, DIM, BN), lambda i, j: (jnp.minimum(i, NB - 1), 0, 0)),
        pl.BlockSpec((NUM_CODES, DIM), lambda i, j: (0, 0)),
    ],
    out_specs=[
        pl.BlockSpec((1, 1, BN), lambda i, j: (jnp.maximum(i - 1, 0), 0, 0)),
        pl.BlockSpec((BN, BK),
                     lambda i, j: (jnp.maximum(i - 1, 0),
                                   jnp.where(i > 0, j, 0))),
        pl.BlockSpec(memory_space=pltpu.SMEM),
    ],
    out_shape=[
        jax.ShapeDtypeStruct((NB, 1, BN), jnp.int32),
        jax.ShapeDtypeStruct((NTOK, NUM_CODES), jnp.float32),
        jax.ShapeDtypeStruct((1, 1), jnp.float32),
    ],
    scratch_shapes=[
        pltpu.VMEM((BN,), jnp.float32),
        pltpu.VMEM((BN,), jnp.int32),
        pltpu.VMEM((BN,), jnp.int32),
        pltpu.SMEM((1, 1), jnp.float32),
    ],
)


_NC, _NS = 2, 16
NW = _NC * _NS
BPW = NTOK // NW
CH = 128
NCH = BPW // CH


def _gather_body(table_hbm, idx_hbm, out_hbm, idx_v, rows_v, sem):
    wid = lax.axis_index("s") * _NC + lax.axis_index("c")
    pltpu.sync_copy(idx_hbm.at[wid], idx_v)
    pltpu.async_copy(table_hbm.at[idx_v.at[0]], rows_v, sem).wait()
    pltpu.sync_copy(rows_v, out_hbm.at[wid])


@functools.cache
def _get_gather_call():
    return pl.kernel(
        _gather_body,
        out_type=jax.ShapeDtypeStruct((NW, CH, DIM), jnp.float32),
        mesh=plsc.VectorSubcoreMesh(core_axis_name="c", subcore_axis_name="s",
                                    num_cores=_NC, num_subcores=_NS),
        scratch_types=[
            pltpu.VMEM((1, CH), jnp.int32),
            pltpu.VMEM((CH, DIM), jnp.float32),
            pltpu.SemaphoreType.DMA,
        ],
    )


def _fin_body(q_ref, qt_ref):
    qt_ref[0] = q_ref[0].T.astype(jnp.bfloat16).astype(jnp.float32)


def _fin_body2(q_ref, qtin_ref, qt_ref):
    del qtin_ref
    qt_ref[0] = q_ref[0].T.astype(jnp.bfloat16).astype(jnp.float32)


_fin_call0 = pl.pallas_call(
    _fin_body,
    grid=(4,),
    in_specs=[
        pl.BlockSpec((1, NTOK // 8, DIM), lambda i: (i, 0, 0)),
    ],
    out_specs=pl.BlockSpec((1, DIM, NTOK // 8), lambda i: (i, 0, 0)),
    out_shape=jax.ShapeDtypeStruct((8, DIM, NTOK // 8), jnp.float32),
)

_fin_call1 = pl.pallas_call(
    _fin_body2,
    grid=(4,),
    in_specs=[
        pl.BlockSpec((1, NTOK // 8, DIM), lambda i: (i, 0, 0)),
        pl.BlockSpec(memory_space=pl.ANY),
    ],
    out_specs=pl.BlockSpec((1, DIM, NTOK // 8), lambda i: (i + 4, 0, 0)),
    out_shape=jax.ShapeDtypeStruct((8, DIM, NTOK // 8), jnp.float32),
    input_output_aliases={1: 0},
)


def kernel(input1, input2_KL, embedding_weight):
    x = jnp.transpose(input1, (0, 2, 3, 1))
    flat = x.reshape(-1, DIM)
    rowsq = jnp.sum(flat**2, axis=1, keepdims=True)
    esq = jnp.sum(embedding_weight**2, axis=1)

    idx3, enc, loss11 = _argmin_call(rowsq.reshape(1, NTOK),
                                     esq.reshape(NUM_CODES, 1),
                                     input1.reshape(8, DIM, NTOK // 8),
                                     embedding_weight)
    idxh = idx3.reshape(2, NW, 1, CH)
    gather = _get_gather_call()
    q0 = gather(embedding_weight, idxh[0])
    qt0 = _fin_call0(q0.reshape(4, NTOK // 8, DIM))
    q1 = gather(embedding_weight, idxh[1])
    qt = _fin_call1(q1.reshape(4, NTOK // 8, DIM), qt0)
    return (
        loss11.reshape(()),
        input2_KL,
        qt.reshape(8, DIM, 32, 32),
        enc,
    )


# speedup vs baseline: 1.0146x; 1.0146x over previous
import functools

import jax
import jax.numpy as jnp
from jax import lax
from jax.experimental import pallas as pl
from jax.experimental.pallas import tpu as pltpu
from jax.experimental.pallas import tpu_sc as plsc

NUM_CODES = 8192
DIM = 256
NTOK = 8192
CCOST = 0.25

BN = 1024
BK = 1024


NB = NTOK // BN
KB = NUM_CODES // BK


def _argmin_body(rowsq_ref, esq_ref, x_ref, e2_ref, idx_ref, enc_ref,
                 loss_ref, min_s, arg_s, prev_s, acc_s):
    i = pl.program_id(0)
    j = pl.program_id(1)

    @pl.when((j == 0) & (i > 0))
    def _():
        prev_s[...] = arg_s[...]

    @pl.when(j == 0)
    def _():
        min_s[...] = jnp.full((BN,), jnp.inf, jnp.float32)
        arg_s[...] = jnp.zeros((BN,), jnp.int32)

    @pl.when(i < NB)
    def _():
        e2 = e2_ref[pl.ds(j * BK, BK), :] * 2.0
        mm2 = jnp.dot(e2, x_ref[0],
                      preferred_element_type=jnp.float32)
        d = (rowsq_ref[...] + esq_ref[...]) - mm2
        lmin = jnp.min(d, axis=0)
        rows = lax.broadcasted_iota(jnp.int32, (BK, BN), 0)
        larg = jnp.min(jnp.where(d == lmin[None, :], rows, BK),
                       axis=0) + j * BK
        better = lmin < min_s[...]
        arg_s[...] = jnp.where(better, larg, arg_s[...])
        min_s[...] = jnp.where(better, lmin, min_s[...])

        @pl.when(j == KB - 1)
        def _():
            s = jnp.sum(min_s[...])
            acc_s[0, 0] = jnp.where(i == 0, s, acc_s[0, 0] + s)

    @pl.when((i == NB) & (j == 0))
    def _():
        m = acc_s[0, 0] / (NTOK * DIM)
        loss_ref[0, 0] = m + CCOST * m

    @pl.when(i > 0)
    def _():
        idv = prev_s[...]
        cols2 = lax.broadcasted_iota(jnp.int32, (BN, BK), 1) + j * BK
        enc_ref[...] = (idv[:, None] == cols2).astype(jnp.float32)

        @pl.when(j == 0)
        def _():
            idx_ref[...] = idv.reshape(1, 1, BN)


_argmin_call = pl.pallas_call(
    _argmin_body,
    grid=(NB + 1, KB),
    in_specs=[
        pl.BlockSpec((1, BN), lambda i, j: (0, jnp.minimum(i, NB - 1))),
        pl.BlockSpec((BK, 1), lambda i, j: (j, 0)),
        pl.BlockSpec((1, DIM, BN), lambda i, j: (jnp.minimum(i, NB - 1), 0, 0)),
        pl.BlockSpec((NUM_CODES, DIM), lambda i, j: (0, 0)),
    ],
    out_specs=[
        pl.BlockSpec((1, 1, BN), lambda i, j: (jnp.maximum(i - 1, 0), 0, 0)),
        pl.BlockSpec((BN, BK),
                     lambda i, j: (jnp.maximum(i - 1, 0),
                                   jnp.where(i > 0, j, 0))),
        pl.BlockSpec(memory_space=pltpu.SMEM),
    ],
    out_shape=[
        jax.ShapeDtypeStruct((NB, 1, BN), jnp.int32),
        jax.ShapeDtypeStruct((NTOK, NUM_CODES), jnp.float32),
        jax.ShapeDtypeStruct((1, 1), jnp.float32),
    ],
    scratch_shapes=[
        pltpu.VMEM((BN,), jnp.float32),
        pltpu.VMEM((BN,), jnp.int32),
        pltpu.VMEM((BN,), jnp.int32),
        pltpu.SMEM((1, 1), jnp.float32),
    ],
)


_NC, _NS = 2, 16
NW = _NC * _NS
BPW = NTOK // NW
CH = 128
NCH = BPW // CH


def _gather_body(table_hbm, idx_hbm, out_hbm, idx_v, rows_v, sem):
    wid = lax.axis_index("s") * _NC + lax.axis_index("c")
    pltpu.sync_copy(idx_hbm.at[wid], idx_v)
    copies = [pltpu.async_copy(table_hbm.at[idx_v.at[c]], rows_v.at[c], sem)
              for c in range(NCH)]
    for cp in copies:
        cp.wait()
    pltpu.sync_copy(rows_v, out_hbm.at[pl.ds(wid * NCH, NCH)])


@functools.cache
def _get_gather_call():
    return pl.kernel(
        _gather_body,
        out_type=jax.ShapeDtypeStruct((NTOK // CH, CH, DIM), jnp.float32),
        mesh=plsc.VectorSubcoreMesh(core_axis_name="c", subcore_axis_name="s",
                                    num_cores=_NC, num_subcores=_NS),
        scratch_types=[
            pltpu.VMEM((NCH, CH), jnp.int32),
            pltpu.VMEM((NCH, CH, DIM), jnp.float32),
            pltpu.SemaphoreType.DMA,
        ],
    )


def _fin_body(q_ref, qt_ref):
    qt_ref[0] = q_ref[0].T.astype(jnp.bfloat16).astype(jnp.float32)


_fin_call = pl.pallas_call(
    _fin_body,
    grid=(8,),
    in_specs=[
        pl.BlockSpec((1, NTOK // 8, DIM), lambda i: (i, 0, 0)),
    ],
    out_specs=pl.BlockSpec((1, DIM, NTOK // 8), lambda i: (i, 0, 0)),
    out_shape=jax.ShapeDtypeStruct((8, DIM, NTOK // 8), jnp.float32),
)


def kernel(input1, input2_KL, embedding_weight):
    x = jnp.transpose(input1, (0, 2, 3, 1))
    flat = x.reshape(-1, DIM)
    rowsq = jnp.sum(flat**2, axis=1, keepdims=True)
    esq = jnp.sum(embedding_weight**2, axis=1)

    idx3, enc, loss11 = _argmin_call(rowsq.reshape(1, NTOK),
                                     esq.reshape(NUM_CODES, 1),
                                     input1.reshape(8, DIM, NTOK // 8),
                                     embedding_weight)
    q = _get_gather_call()(embedding_weight,
                           idx3.reshape(NW, NCH, CH))
    qt = _fin_call(q.reshape(8, NTOK // 8, DIM))
    return (
        loss11.reshape(()),
        input2_KL,
        qt.reshape(8, DIM, 32, 32),
        enc,
    )


# speedup vs baseline: 1.0736x; 1.0581x over previous
import functools

import jax
import jax.numpy as jnp
from jax import lax
from jax.experimental import pallas as pl
from jax.experimental.pallas import tpu as pltpu
from jax.experimental.pallas import tpu_sc as plsc

NUM_CODES = 8192
DIM = 256
NTOK = 8192
CCOST = 0.25

BN = 1024
BK = 2048


NB = NTOK // BN
KB = NUM_CODES // BK


def _argmin_body(rowsq_ref, esq_ref, x_ref, e2_ref, idx_ref, enc_ref,
                 loss_ref, min_s, arg_s, prev_s, acc_s):
    i = pl.program_id(0)
    j = pl.program_id(1)

    @pl.when((j == 0) & (i > 0))
    def _():
        prev_s[...] = arg_s[...]

    @pl.when(j == 0)
    def _():
        min_s[...] = jnp.full((BN,), jnp.inf, jnp.float32)
        arg_s[...] = jnp.zeros((BN,), jnp.int32)

    @pl.when(i < NB)
    def _():
        e2 = e2_ref[pl.ds(j * BK, BK), :] * 2.0
        mm2 = jnp.dot(e2, x_ref[0],
                      preferred_element_type=jnp.float32)
        d = (rowsq_ref[...] + esq_ref[...]) - mm2
        lmin = jnp.min(d, axis=0)
        rows = lax.broadcasted_iota(jnp.int32, (BK, BN), 0)
        larg = jnp.min(jnp.where(d == lmin[None, :], rows, BK),
                       axis=0) + j * BK
        better = lmin < min_s[...]
        arg_s[...] = jnp.where(better, larg, arg_s[...])
        min_s[...] = jnp.where(better, lmin, min_s[...])

        @pl.when(j == KB - 1)
        def _():
            s = jnp.sum(min_s[...])
            acc_s[0, 0] = jnp.where(i == 0, s, acc_s[0, 0] + s)

    @pl.when((i == NB) & (j == 0))
    def _():
        m = acc_s[0, 0] / (NTOK * DIM)
        loss_ref[0, 0] = m + CCOST * m

    @pl.when(i > 0)
    def _():
        idv = prev_s[...]
        cols2 = lax.broadcasted_iota(jnp.int32, (BN, BK), 1) + j * BK
        enc_ref[...] = (idv[:, None] == cols2).astype(jnp.float32)

        @pl.when(j == 0)
        def _():
            idx_ref[...] = idv.reshape(1, 1, BN)


_argmin_call = pl.pallas_call(
    _argmin_body,
    grid=(NB + 1, KB),
    in_specs=[
        pl.BlockSpec((1, BN), lambda i, j: (0, jnp.minimum(i, NB - 1))),
        pl.BlockSpec((BK, 1), lambda i, j: (j, 0)),
        pl.BlockSpec((1, DIM, BN), lambda i, j: (jnp.minimum(i, NB - 1), 0, 0)),
        pl.BlockSpec((NUM_CODES, DIM), lambda i, j: (0, 0)),
    ],
    out_specs=[
        pl.BlockSpec((1, 1, BN), lambda i, j: (jnp.maximum(i - 1, 0), 0, 0)),
        pl.BlockSpec((BN, BK),
                     lambda i, j: (jnp.maximum(i - 1, 0),
                                   jnp.where(i > 0, j, 0))),
        pl.BlockSpec(memory_space=pltpu.SMEM),
    ],
    out_shape=[
        jax.ShapeDtypeStruct((NB, 1, BN), jnp.int32),
        jax.ShapeDtypeStruct((NTOK, NUM_CODES), jnp.float32),
        jax.ShapeDtypeStruct((1, 1), jnp.float32),
    ],
    scratch_shapes=[
        pltpu.VMEM((BN,), jnp.float32),
        pltpu.VMEM((BN,), jnp.int32),
        pltpu.VMEM((BN,), jnp.int32),
        pltpu.SMEM((1, 1), jnp.float32),
    ],
)


_NC, _NS = 2, 16
NW = _NC * _NS
BPW = NTOK // NW
CH = 128
NCH = BPW // CH


def _gather_body(table_hbm, idx_hbm, out_hbm, idx_v, rows_v, sem):
    wid = lax.axis_index("s") * _NC + lax.axis_index("c")
    pltpu.sync_copy(idx_hbm.at[wid], idx_v)
    copies = [pltpu.async_copy(table_hbm.at[idx_v.at[c]], rows_v.at[c], sem)
              for c in range(NCH)]
    for cp in copies:
        cp.wait()
    pltpu.sync_copy(rows_v, out_hbm.at[pl.ds(wid * NCH, NCH)])


@functools.cache
def _get_gather_call():
    return pl.kernel(
        _gather_body,
        out_type=jax.ShapeDtypeStruct((NTOK // CH, CH, DIM), jnp.float32),
        mesh=plsc.VectorSubcoreMesh(core_axis_name="c", subcore_axis_name="s",
                                    num_cores=_NC, num_subcores=_NS),
        scratch_types=[
            pltpu.VMEM((NCH, CH), jnp.int32),
            pltpu.VMEM((NCH, CH, DIM), jnp.float32),
            pltpu.SemaphoreType.DMA,
        ],
    )


def _fin_body(q_ref, qt_ref):
    qt_ref[0] = q_ref[0].T.astype(jnp.bfloat16).astype(jnp.float32)


_fin_call = pl.pallas_call(
    _fin_body,
    grid=(8,),
    in_specs=[
        pl.BlockSpec((1, NTOK // 8, DIM), lambda i: (i, 0, 0)),
    ],
    out_specs=pl.BlockSpec((1, DIM, NTOK // 8), lambda i: (i, 0, 0)),
    out_shape=jax.ShapeDtypeStruct((8, DIM, NTOK // 8), jnp.float32),
)


def kernel(input1, input2_KL, embedding_weight):
    x = jnp.transpose(input1, (0, 2, 3, 1))
    flat = x.reshape(-1, DIM)
    rowsq = jnp.sum(flat**2, axis=1, keepdims=True)
    esq = jnp.sum(embedding_weight**2, axis=1)

    idx3, enc, loss11 = _argmin_call(rowsq.reshape(1, NTOK),
                                     esq.reshape(NUM_CODES, 1),
                                     input1.reshape(8, DIM, NTOK // 8),
                                     embedding_weight)
    q = _get_gather_call()(embedding_weight,
                           idx3.reshape(NW, NCH, CH))
    qt = _fin_call(q.reshape(8, NTOK // 8, DIM))
    return (
        loss11.reshape(()),
        input2_KL,
        qt.reshape(8, DIM, 32, 32),
        enc,
    )
